# baseline (speedup 1.0000x reference)
import jax
import jax.numpy as jnp
from jax.experimental import pallas as pl
from jax.experimental.pallas import tpu as pltpu

_ROW_BLK = 512


def _add_kernel(x_ref, pos_ref, o_ref):
    o_ref[...] = x_ref[...] + pos_ref[...]


def kernel(x, pos_table):
    batch, maxlen, embed = x.shape
    x2 = x.reshape(batch * maxlen, embed)
    blocks_per_seq = maxlen // _ROW_BLK
    grid = (batch * maxlen // _ROW_BLK,)
    out = pl.pallas_call(
        _add_kernel,
        grid=grid,
        in_specs=[
            pl.BlockSpec((_ROW_BLK, embed), lambda i: (i, 0)),
            pl.BlockSpec((_ROW_BLK, embed), lambda i: (i % blocks_per_seq, 0)),
        ],
        out_specs=pl.BlockSpec((_ROW_BLK, embed), lambda i: (i, 0)),
        out_shape=jax.ShapeDtypeStruct(x2.shape, x2.dtype),
        compiler_params=pltpu.CompilerParams(
            dimension_semantics=("arbitrary",),
        ),
    )(x2, pos_table)
    return out.reshape(x.shape)
